# baseline (device time: 201618 ns/iter reference)
import jax
import jax.numpy as jnp
from jax import lax
from jax.experimental import pallas as pl
from jax.experimental.pallas import tpu as pltpu

N_DEV = 32
B, S, H, Dh, Dr = 2, 256, 16, 64, 32
D = 1024
ROWS = B * S
KV_W = 2 * D
CHUNK = ROWS // N_DEV


def kernel(x, Wdkv, Wuk, Wuv, Wq, Wqr, Wkr, Wo):
    x2 = x.reshape(ROWS, D)

    def body(x_ref, wdkv_ref, wuk_ref, wuv_ref, wq_ref, wqr_ref, wkr_ref,
             wo_ref, out_ref, acc_ref, stage_rs, stage_ag,
             rs_send, rs_recv, ag_send, ag_recv):
        my = lax.axis_index("i")
        left = lax.rem(my + N_DEV - 1, N_DEV)
        right = lax.rem(my + 1, N_DEV)

        xb = x_ref[...].astype(jnp.bfloat16)

        c = jnp.dot(xb, wdkv_ref[...].astype(jnp.bfloat16),
                    preferred_element_type=jnp.float32).astype(jnp.bfloat16)
        kp = jnp.dot(c, wuk_ref[...].astype(jnp.bfloat16),
                     preferred_element_type=jnp.float32)
        vp = jnp.dot(c, wuv_ref[...].astype(jnp.bfloat16),
                     preferred_element_type=jnp.float32)
        acc_ref[:, :D] = kp.astype(jnp.bfloat16)
        acc_ref[:, D:] = vp.astype(jnp.bfloat16)

        q = jnp.dot(xb, wq_ref[...].astype(jnp.bfloat16),
                    preferred_element_type=jnp.float32).astype(jnp.bfloat16)
        qr = jnp.dot(xb, wqr_ref[...].astype(jnp.bfloat16),
                     preferred_element_type=jnp.float32).astype(jnp.bfloat16)
        kr = jnp.dot(xb, wkr_ref[...].astype(jnp.bfloat16),
                     preferred_element_type=jnp.float32).astype(jnp.bfloat16)

        barrier = pltpu.get_barrier_semaphore()
        pl.semaphore_signal(barrier, inc=1, device_id=(left,),
                            device_id_type=pl.DeviceIdType.MESH)
        pl.semaphore_signal(barrier, inc=1, device_id=(right,),
                            device_id_type=pl.DeviceIdType.MESH)
        pl.semaphore_wait(barrier, 2)

        def chunk_rows(idx):
            return pl.ds(idx * CHUNK, CHUNK)

        for t in range(N_DEV - 1):
            send_idx = lax.rem(my - t + 2 * N_DEV, N_DEV)
            recv_idx = lax.rem(my - t - 1 + 2 * N_DEV, N_DEV)
            rdma = pltpu.make_async_remote_copy(
                src_ref=acc_ref.at[chunk_rows(send_idx), :],
                dst_ref=stage_rs.at[t],
                send_sem=rs_send.at[t],
                recv_sem=rs_recv.at[t],
                device_id=(right,),
                device_id_type=pl.DeviceIdType.MESH,
            )
            rdma.start()
            rdma.wait()
            acc_ref[chunk_rows(recv_idx), :] = (
                acc_ref[chunk_rows(recv_idx), :] + stage_rs[t])

        owned = lax.rem(my + 1, N_DEV)

        for t in range(N_DEV - 1):
            src = acc_ref.at[chunk_rows(owned), :] if t == 0 else stage_ag.at[t - 1]
            rdma = pltpu.make_async_remote_copy(
                src_ref=src,
                dst_ref=stage_ag.at[t],
                send_sem=ag_send.at[t],
                recv_sem=ag_recv.at[t],
                device_id=(right,),
                device_id_type=pl.DeviceIdType.MESH,
            )
            rdma.start()
            rdma.wait()
            got = lax.rem(my - t + 2 * N_DEV, N_DEV)
            acc_ref[chunk_rows(got), :] = stage_ag[t]

        scale = (Dh + Dr) ** -0.5
        nt = (((1,), (1,)), ((), ()))
        for b in range(B):
            r0 = b * S
            kr_b = kr[r0:r0 + S, :]
            o_b = jnp.zeros((S, D), jnp.float32)
            for h in range(H):
                q_bh = q[r0:r0 + S, h * Dh:(h + 1) * Dh]
                qr_bh = qr[r0:r0 + S, h * Dr:(h + 1) * Dr]
                k_bh = acc_ref[r0:r0 + S, h * Dh:(h + 1) * Dh]
                v_bh = acc_ref[r0:r0 + S, D + h * Dh:D + (h + 1) * Dh]
                scores = (
                    lax.dot_general(q_bh, k_bh, nt,
                                    preferred_element_type=jnp.float32)
                    + lax.dot_general(qr_bh, kr_b, nt,
                                      preferred_element_type=jnp.float32)
                ) * scale
                m = jnp.max(scores, axis=-1, keepdims=True)
                p = jnp.exp(scores - m)
                p = p / jnp.sum(p, axis=-1, keepdims=True)
                o_bh = jnp.dot(p.astype(jnp.bfloat16), v_bh,
                               preferred_element_type=jnp.float32)
                o_b = o_b + jnp.dot(
                    o_bh.astype(jnp.bfloat16),
                    wo_ref[h * Dh:(h + 1) * Dh, :].astype(jnp.bfloat16),
                    preferred_element_type=jnp.float32)
            out_ref[r0:r0 + S, :] = o_b

    out = pl.pallas_call(
        body,
        out_shape=jax.ShapeDtypeStruct((ROWS, D), jnp.float32),
        in_specs=[pl.BlockSpec(memory_space=pltpu.VMEM)] * 8,
        out_specs=pl.BlockSpec(memory_space=pltpu.VMEM),
        scratch_shapes=[
            pltpu.VMEM((ROWS, KV_W), jnp.bfloat16),
            pltpu.VMEM((N_DEV - 1, CHUNK, KV_W), jnp.bfloat16),
            pltpu.VMEM((N_DEV - 1, CHUNK, KV_W), jnp.bfloat16),
            pltpu.SemaphoreType.DMA((N_DEV - 1,)),
            pltpu.SemaphoreType.DMA((N_DEV - 1,)),
            pltpu.SemaphoreType.DMA((N_DEV - 1,)),
            pltpu.SemaphoreType.DMA((N_DEV - 1,)),
        ],
        compiler_params=pltpu.CompilerParams(collective_id=0),
    )(x2, Wdkv, Wuk, Wuv, Wq, Wqr, Wkr, Wo)

    return out.reshape(B, S, D)


# device time: 103586 ns/iter; 1.9464x vs baseline; 1.9464x over previous
import jax
import jax.numpy as jnp
from jax import lax
from jax.experimental import pallas as pl
from jax.experimental.pallas import tpu as pltpu

N_DEV = 32
B, S, H, Dh, Dr = 2, 256, 16, 64, 32
D = 1024
ROWS = B * S
KV_W = 2 * D
CHUNK = ROWS // N_DEV


def kernel(x, Wdkv, Wuk, Wuv, Wq, Wqr, Wkr, Wo):
    x2 = x.reshape(ROWS, D)

    def body(x_ref, wdkv_ref, wuk_ref, wuv_ref, wq_ref, wqr_ref, wkr_ref,
             wo_ref, out_ref, acc_ref, st0, st1, st2, st3, st4,
             rs_send, rs_recv, ag_send, ag_recv):
        stages = [st0, st1, st2, st3, st4]
        my = lax.axis_index("i")

        z = my // 8
        o = my % 8
        y = o // 2
        x_ = (y + o) % 2

        def lof(px, py, pz):
            return pz * 8 + py * 2 + (px + py) % 2

        partners = [
            lof(1 - x_, y, z),
            lof(x_, y + 1 - 2 * (y % 2), z),
            lof(x_, y, z + 1 - 2 * (z % 2)),
            lof(x_, (y + 2) % 4, z),
            lof(x_, y, (z + 2) % 4),
        ]
        p = x_ * 16 + (y % 2) * 8 + (z % 2) * 4 + (y // 2) * 2 + z // 2

        xb = x_ref[...].astype(jnp.bfloat16)

        c = jnp.dot(xb, wdkv_ref[...].astype(jnp.bfloat16),
                    preferred_element_type=jnp.float32).astype(jnp.bfloat16)
        kp = jnp.dot(c, wuk_ref[...].astype(jnp.bfloat16),
                     preferred_element_type=jnp.float32)
        vp = jnp.dot(c, wuv_ref[...].astype(jnp.bfloat16),
                     preferred_element_type=jnp.float32)
        acc_ref[:, :D] = kp.astype(jnp.bfloat16)
        acc_ref[:, D:] = vp.astype(jnp.bfloat16)

        barrier = pltpu.get_barrier_semaphore()
        for prt in partners:
            pl.semaphore_signal(barrier, inc=1, device_id=(prt,),
                                device_id_type=pl.DeviceIdType.MESH)
        pl.semaphore_wait(barrier, len(partners))

        q_proj = qr_proj = kr_proj = None
        for k in range(5):
            h = 16 >> k
            rows = h * CHUNK
            qq = p // h
            sign = 1 - 2 * (qq % 2)
            kept = qq * rows
            sent = (qq + sign) * rows
            rdma = pltpu.make_async_remote_copy(
                src_ref=acc_ref.at[pl.ds(sent, rows), :],
                dst_ref=stages[k].at[...],
                send_sem=rs_send.at[k],
                recv_sem=rs_recv.at[k],
                device_id=(partners[k],),
                device_id_type=pl.DeviceIdType.MESH,
            )
            rdma.start()
            if k == 0:
                q_proj = jnp.dot(xb, wq_ref[...].astype(jnp.bfloat16),
                                 preferred_element_type=jnp.float32
                                 ).astype(jnp.bfloat16)
                qr_proj = jnp.dot(xb, wqr_ref[...].astype(jnp.bfloat16),
                                  preferred_element_type=jnp.float32
                                  ).astype(jnp.bfloat16)
                kr_proj = jnp.dot(xb, wkr_ref[...].astype(jnp.bfloat16),
                                  preferred_element_type=jnp.float32
                                  ).astype(jnp.bfloat16)
            rdma.wait()
            acc_ref[pl.ds(kept, rows), :] = (
                acc_ref[pl.ds(kept, rows), :] + stages[k][...])

        for j in range(5):
            blk = 1 << j
            rows = blk * CHUNK
            qb = p // blk
            sign = 1 - 2 * (qb % 2)
            own = qb * rows
            recv = (qb + sign) * rows
            rdma = pltpu.make_async_remote_copy(
                src_ref=acc_ref.at[pl.ds(own, rows), :],
                dst_ref=acc_ref.at[pl.ds(own, rows), :],
                send_sem=ag_send.at[j],
                recv_sem=ag_recv.at[j],
                device_id=(partners[4 - j],),
                device_id_type=pl.DeviceIdType.MESH,
            )
            rdma.start()
            rdma.wait()
            del recv

        scale = (Dh + Dr) ** -0.5
        nt = (((1,), (1,)), ((), ()))
        for b in range(B):
            r0 = b * S
            kr_b = kr_proj[r0:r0 + S, :]
            o_b = jnp.zeros((S, D), jnp.float32)
            for hh in range(H):
                q_bh = q_proj[r0:r0 + S, hh * Dh:(hh + 1) * Dh]
                qr_bh = qr_proj[r0:r0 + S, hh * Dr:(hh + 1) * Dr]
                k_bh = acc_ref[r0:r0 + S, hh * Dh:(hh + 1) * Dh]
                v_bh = acc_ref[r0:r0 + S, D + hh * Dh:D + (hh + 1) * Dh]
                scores = (
                    lax.dot_general(q_bh, k_bh, nt,
                                    preferred_element_type=jnp.float32)
                    + lax.dot_general(qr_bh, kr_b, nt,
                                      preferred_element_type=jnp.float32)
                ) * scale
                m = jnp.max(scores, axis=-1, keepdims=True)
                pr = jnp.exp(scores - m)
                pr = pr / jnp.sum(pr, axis=-1, keepdims=True)
                o_bh = jnp.dot(pr.astype(jnp.bfloat16), v_bh,
                               preferred_element_type=jnp.float32)
                o_b = o_b + jnp.dot(
                    o_bh.astype(jnp.bfloat16),
                    wo_ref[hh * Dh:(hh + 1) * Dh, :].astype(jnp.bfloat16),
                    preferred_element_type=jnp.float32)
            out_ref[r0:r0 + S, :] = o_b

    out = pl.pallas_call(
        body,
        out_shape=jax.ShapeDtypeStruct((ROWS, D), jnp.float32),
        in_specs=[pl.BlockSpec(memory_space=pltpu.VMEM)] * 8,
        out_specs=pl.BlockSpec(memory_space=pltpu.VMEM),
        scratch_shapes=[
            pltpu.VMEM((ROWS, KV_W), jnp.bfloat16),
            pltpu.VMEM((256, KV_W), jnp.bfloat16),
            pltpu.VMEM((128, KV_W), jnp.bfloat16),
            pltpu.VMEM((64, KV_W), jnp.bfloat16),
            pltpu.VMEM((32, KV_W), jnp.bfloat16),
            pltpu.VMEM((16, KV_W), jnp.bfloat16),
            pltpu.SemaphoreType.DMA((5,)),
            pltpu.SemaphoreType.DMA((5,)),
            pltpu.SemaphoreType.DMA((5,)),
            pltpu.SemaphoreType.DMA((5,)),
        ],
        compiler_params=pltpu.CompilerParams(collective_id=0),
    )(x2, Wdkv, Wuk, Wuv, Wq, Wqr, Wkr, Wo)

    return out.reshape(B, S, D)


# device time: 86422 ns/iter; 2.3329x vs baseline; 1.1986x over previous
import jax
import jax.numpy as jnp
from jax import lax
from jax.experimental import pallas as pl
from jax.experimental.pallas import tpu as pltpu

N_DEV = 32
B, S, H, Dh, Dr = 2, 256, 16, 64, 32
D = 1024
ROWS = B * S
CHUNK = ROWS // N_DEV


def kernel(x, Wdkv, Wuk, Wuv, Wq, Wqr, Wkr, Wo):
    x2 = x.reshape(ROWS, D)

    def body(x_ref, wdkv_ref, wuk_ref, wuv_ref, wq_ref, wqr_ref, wkr_ref,
             wo_ref, out_ref, acc_a, acc_b,
             sa0, sa1, sa2, sa3, sa4, sb0, sb1, sb2, sb3, sb4,
             rs_send, rs_recv, ag_send, ag_recv):
        st_a = [sa0, sa1, sa2, sa3, sa4]
        st_b = [sb0, sb1, sb2, sb3, sb4]
        my = lax.axis_index("i")

        z = my // 8
        o = my % 8
        y = o // 2
        x_ = (y + o) % 2

        def lof(px, py, pz):
            return pz * 8 + py * 2 + (px + py) % 2

        partners = [
            lof(1 - x_, y, z),
            lof(x_, y + 1 - 2 * (y % 2), z),
            lof(x_, y, z + 1 - 2 * (z % 2)),
            lof(x_, (y + 2) % 4, z),
            lof(x_, y, (z + 2) % 4),
        ]
        p = x_ * 16 + (y % 2) * 8 + (z % 2) * 4 + (y // 2) * 2 + z // 2

        def rs_off(k):
            h = 16 >> k
            rows = h * CHUNK
            qq = p // h
            sign = 1 - 2 * (qq % 2)
            return qq * rows, (qq + sign) * rows, rows

        def ag_off(j):
            blk = 1 << j
            rows = blk * CHUNK
            return (p // blk) * rows, rows

        def rs_rdma(acc, st, k, sem_off):
            kept, sent, rows = rs_off(k)
            return pltpu.make_async_remote_copy(
                src_ref=acc.at[pl.ds(sent, rows), :],
                dst_ref=st[k].at[...],
                send_sem=rs_send.at[sem_off + k],
                recv_sem=rs_recv.at[sem_off + k],
                device_id=(partners[k],),
                device_id_type=pl.DeviceIdType.MESH,
            )

        def rs_acc(acc, st, k):
            kept, _, rows = rs_off(k)
            acc[pl.ds(kept, rows), :] = acc[pl.ds(kept, rows), :] + st[k][...]

        def ag_rdma(acc, j, sem_off):
            own, rows = ag_off(j)
            return pltpu.make_async_remote_copy(
                src_ref=acc.at[pl.ds(own, rows), :],
                dst_ref=acc.at[pl.ds(own, rows), :],
                send_sem=ag_send.at[sem_off + j],
                recv_sem=ag_recv.at[sem_off + j],
                device_id=(partners[4 - j],),
                device_id_type=pl.DeviceIdType.MESH,
            )

        xb = x_ref[...].astype(jnp.bfloat16)

        c = jnp.dot(xb, wdkv_ref[...].astype(jnp.bfloat16),
                    preferred_element_type=jnp.float32).astype(jnp.bfloat16)
        kp = jnp.dot(c, wuk_ref[...].astype(jnp.bfloat16),
                     preferred_element_type=jnp.float32)
        vp = jnp.dot(c, wuv_ref[...].astype(jnp.bfloat16),
                     preferred_element_type=jnp.float32)
        acc_a[...] = kp.astype(jnp.bfloat16)
        acc_b[...] = vp.astype(jnp.bfloat16)

        barrier = pltpu.get_barrier_semaphore()
        for prt in partners:
            pl.semaphore_signal(barrier, inc=1, device_id=(prt,),
                                device_id_type=pl.DeviceIdType.MESH)
        pl.semaphore_wait(barrier, len(partners))

        ra = rs_rdma(acc_a, st_a, 0, 0)
        ra.start()
        q_proj = jnp.dot(xb, wq_ref[...].astype(jnp.bfloat16),
                         preferred_element_type=jnp.float32).astype(jnp.bfloat16)
        qr_proj = jnp.dot(xb, wqr_ref[...].astype(jnp.bfloat16),
                          preferred_element_type=jnp.float32).astype(jnp.bfloat16)
        kr_proj = jnp.dot(xb, wkr_ref[...].astype(jnp.bfloat16),
                          preferred_element_type=jnp.float32).astype(jnp.bfloat16)
        ra.wait()
        rs_acc(acc_a, st_a, 0)

        for k in range(1, 5):
            ra = rs_rdma(acc_a, st_a, k, 0)
            rb = rs_rdma(acc_b, st_b, k - 1, 5)
            ra.start()
            rb.start()
            ra.wait()
            rs_acc(acc_a, st_a, k)
            rb.wait()
            rs_acc(acc_b, st_b, k - 1)

        ga = ag_rdma(acc_a, 0, 0)
        rb = rs_rdma(acc_b, st_b, 4, 5)
        ga.start()
        rb.start()
        ga.wait()
        rb.wait()
        rs_acc(acc_b, st_b, 4)

        for j in range(1, 5):
            ga = ag_rdma(acc_a, j, 0)
            gb = ag_rdma(acc_b, j - 1, 5)
            ga.start()
            gb.start()
            ga.wait()
            gb.wait()

        gb = ag_rdma(acc_b, 4, 5)
        gb.start()

        scale = (Dh + Dr) ** -0.5
        nt = (((1,), (1,)), ((), ()))
        probs = []
        for b in range(B):
            r0 = b * S
            kr_b = kr_proj[r0:r0 + S, :]
            for hh in range(H):
                q_bh = q_proj[r0:r0 + S, hh * Dh:(hh + 1) * Dh]
                qr_bh = qr_proj[r0:r0 + S, hh * Dr:(hh + 1) * Dr]
                k_bh = acc_a[r0:r0 + S, hh * Dh:(hh + 1) * Dh]
                scores = (
                    lax.dot_general(q_bh, k_bh, nt,
                                    preferred_element_type=jnp.float32)
                    + lax.dot_general(qr_bh, kr_b, nt,
                                      preferred_element_type=jnp.float32)
                ) * scale
                m = jnp.max(scores, axis=-1, keepdims=True)
                pr = jnp.exp(scores - m)
                pr = pr / jnp.sum(pr, axis=-1, keepdims=True)
                probs.append(pr.astype(jnp.bfloat16))

        gb.wait()

        for b in range(B):
            r0 = b * S
            o_b = jnp.zeros((S, D), jnp.float32)
            for hh in range(H):
                v_bh = acc_b[r0:r0 + S, hh * Dh:(hh + 1) * Dh]
                o_bh = jnp.dot(probs[b * H + hh], v_bh,
                               preferred_element_type=jnp.float32)
                o_b = o_b + jnp.dot(
                    o_bh.astype(jnp.bfloat16),
                    wo_ref[hh * Dh:(hh + 1) * Dh, :].astype(jnp.bfloat16),
                    preferred_element_type=jnp.float32)
            out_ref[r0:r0 + S, :] = o_b

    stage_shapes = [pltpu.VMEM((r, D), jnp.bfloat16)
                    for r in (256, 128, 64, 32, 16)] * 2

    out = pl.pallas_call(
        body,
        out_shape=jax.ShapeDtypeStruct((ROWS, D), jnp.float32),
        in_specs=[pl.BlockSpec(memory_space=pltpu.VMEM)] * 8,
        out_specs=pl.BlockSpec(memory_space=pltpu.VMEM),
        scratch_shapes=[
            pltpu.VMEM((ROWS, D), jnp.bfloat16),
            pltpu.VMEM((ROWS, D), jnp.bfloat16),
            *stage_shapes,
            pltpu.SemaphoreType.DMA((10,)),
            pltpu.SemaphoreType.DMA((10,)),
            pltpu.SemaphoreType.DMA((10,)),
            pltpu.SemaphoreType.DMA((10,)),
        ],
        compiler_params=pltpu.CompilerParams(collective_id=0),
    )(x2, Wdkv, Wuk, Wuv, Wq, Wqr, Wkr, Wo)

    return out.reshape(B, S, D)


# device time: 73964 ns/iter; 2.7259x vs baseline; 1.1684x over previous
import jax
import jax.numpy as jnp
from jax import lax
from jax.experimental import pallas as pl
from jax.experimental.pallas import tpu as pltpu

N_DEV = 32
B, S, H, Dh, Dr = 2, 256, 16, 64, 32
D = 1024
ROWS = B * S
CHUNK = ROWS // N_DEV


def kernel(x, Wdkv, Wuk, Wuv, Wq, Wqr, Wkr, Wo):
    x2 = x.reshape(ROWS, D)

    def body(x_ref, wdkv_ref, wuk_ref, wuv_ref, wq_ref, wqr_ref, wkr_ref,
             wo_ref, out_ref, acc_a, acc_b,
             la0, la1, la2, lb0, lb1, lb2,
             qa0, qa1, qa2, qb0, qb1, qb2,
             q_ref, qr_ref, kr_ref, probs_ref,
             rs_send, rs_recv, ag_send, ag_recv,
             qrs_send, qrs_recv, qag_send, qag_recv):
        my = lax.axis_index("i")

        z = my // 8
        o = my % 8
        y = o // 2
        x_ = (y + o) % 2

        def lof(px, py, pz):
            return pz * 8 + py * 2 + (px + py) % 2

        y_1 = y + 1 - 2 * (y % 2)
        z_1 = z + 1 - 2 * (z % 2)
        y_2 = (y + 2) % 4
        z_2 = (z + 2) % 4
        z_3 = (z_1 + 2) % 4

        pA = x_ * 16 + (y % 2) * 8 + (z % 2) * 4 + (y // 2) * 2 + z // 2
        A_lin = [lof(1 - x_, y, z), lof(x_, y_1, z), lof(x_, y, z_1)]
        subA = (y // 2) * 2 + z // 2
        A_quad = [
            (lof(x_, y_2, z), (y_2 // 2) * 2 + z // 2),
            (lof(x_, y, z_2), (y // 2) * 2 + z_2 // 2),
            (lof(x_, y_2, z_2), (y_2 // 2) * 2 + z_2 // 2),
        ]

        pB = (y % 2) * 16 + x_ * 8 + (y // 2) * 4 + (z % 2) * 2 + z // 2
        B_lin = [lof(x_, y_1, z), lof(1 - x_, y, z), lof(x_, y_2, z)]
        subB = (z % 2) * 2 + z // 2
        B_quad = [
            (lof(x_, y, z_1), (z_1 % 2) * 2 + z_1 // 2),
            (lof(x_, y, z_2), (z_2 % 2) * 2 + z_2 // 2),
            (lof(x_, y, z_3), (z_3 % 2) * 2 + z_3 // 2),
        ]

        streams = [
            dict(acc=acc_a, p=pA, lin=A_lin, lst=[la0, la1, la2],
                 qst=[qa0, qa1, qa2], quad=A_quad, sub=subA, so=0),
            dict(acc=acc_b, p=pB, lin=B_lin, lst=[lb0, lb1, lb2],
                 qst=[qb0, qb1, qb2], quad=B_quad, sub=subB, so=3),
        ]

        def lin_rs_rdma(st, k):
            h = 16 >> k
            rows = h * CHUNK
            qq = st["p"] // h
            sent = (qq + 1 - 2 * (qq % 2)) * rows
            return pltpu.make_async_remote_copy(
                src_ref=st["acc"].at[pl.ds(sent, rows), :],
                dst_ref=st["lst"][k].at[...],
                send_sem=rs_send.at[st["so"] + k],
                recv_sem=rs_recv.at[st["so"] + k],
                device_id=(st["lin"][k],),
                device_id_type=pl.DeviceIdType.MESH,
            )

        def lin_rs_acc(st, k):
            h = 16 >> k
            rows = h * CHUNK
            kept = (st["p"] // h) * rows
            st["acc"][pl.ds(kept, rows), :] = (
                st["acc"][pl.ds(kept, rows), :] + st["lst"][k][...])

        def lin_ag_rdma(st, j):
            blk = 1 << j
            rows = blk * CHUNK
            own = (st["p"] // blk) * rows
            return pltpu.make_async_remote_copy(
                src_ref=st["acc"].at[pl.ds(own, rows), :],
                dst_ref=st["acc"].at[pl.ds(own, rows), :],
                send_sem=ag_send.at[st["so"] + (4 - j)],
                recv_sem=ag_recv.at[st["so"] + (4 - j)],
                device_id=(st["lin"][4 - j],),
                device_id_type=pl.DeviceIdType.MESH,
            )

        def quad_rs_rdmas(st):
            q4 = st["p"] // 4
            return [pltpu.make_async_remote_copy(
                src_ref=st["acc"].at[pl.ds((q4 * 4 + ps) * CHUNK, CHUNK), :],
                dst_ref=st["qst"][i].at[...],
                send_sem=qrs_send.at[st["so"] + i],
                recv_sem=qrs_recv.at[st["so"] + i],
                device_id=(prt,),
                device_id_type=pl.DeviceIdType.MESH,
            ) for i, (prt, ps) in enumerate(st["quad"])]

        def quad_ag_rdmas(st):
            own = pl.ds(st["p"] * CHUNK, CHUNK)
            return [pltpu.make_async_remote_copy(
                src_ref=st["acc"].at[own, :],
                dst_ref=st["acc"].at[own, :],
                send_sem=qag_send.at[st["so"] + i],
                recv_sem=qag_recv.at[st["so"] + i],
                device_id=(prt,),
                device_id_type=pl.DeviceIdType.MESH,
            ) for i, (prt, _) in enumerate(st["quad"])]

        xb = x_ref[...].astype(jnp.bfloat16)

        c = jnp.dot(xb, wdkv_ref[...].astype(jnp.bfloat16),
                    preferred_element_type=jnp.float32).astype(jnp.bfloat16)
        kp = jnp.dot(c, wuk_ref[...].astype(jnp.bfloat16),
                     preferred_element_type=jnp.float32)
        vp = jnp.dot(c, wuv_ref[...].astype(jnp.bfloat16),
                     preferred_element_type=jnp.float32)
        acc_a[...] = kp.astype(jnp.bfloat16)
        acc_b[...] = vp.astype(jnp.bfloat16)

        bar_partners = [lof(1 - x_, y, z), lof(x_, y_1, z), lof(x_, y, z_1),
                        lof(x_, y_2, z), lof(x_, y, z_2),
                        lof(x_, y_2, z_2), lof(x_, y, z_3)]
        barrier = pltpu.get_barrier_semaphore()
        for prt in bar_partners:
            pl.semaphore_signal(barrier, inc=1, device_id=(prt,),
                                device_id_type=pl.DeviceIdType.MESH)
        pl.semaphore_wait(barrier, len(bar_partners))

        ra = lin_rs_rdma(streams[0], 0)
        rb = lin_rs_rdma(streams[1], 0)
        ra.start()
        rb.start()
        q_ref[...] = jnp.dot(xb, wq_ref[...].astype(jnp.bfloat16),
                             preferred_element_type=jnp.float32
                             ).astype(jnp.bfloat16)
        qr_ref[...] = jnp.dot(xb, wqr_ref[...].astype(jnp.bfloat16),
                              preferred_element_type=jnp.float32
                              ).astype(jnp.bfloat16)
        kr_ref[...] = jnp.dot(xb, wkr_ref[...].astype(jnp.bfloat16),
                              preferred_element_type=jnp.float32
                              ).astype(jnp.bfloat16)
        ra.wait()
        lin_rs_acc(streams[0], 0)
        rb.wait()
        lin_rs_acc(streams[1], 0)

        for k in (1, 2):
            ra = lin_rs_rdma(streams[0], k)
            rb = lin_rs_rdma(streams[1], k)
            ra.start()
            rb.start()
            ra.wait()
            lin_rs_acc(streams[0], k)
            rb.wait()
            lin_rs_acc(streams[1], k)

        qra = quad_rs_rdmas(streams[0])
        qrb = quad_rs_rdmas(streams[1])
        for r in qra + qrb:
            r.start()
        for r in qra:
            r.wait()
        ownA = pl.ds(pA * CHUNK, CHUNK)
        acc_a[ownA, :] = (acc_a[ownA, :] + qa0[...] + qa1[...] + qa2[...])
        for r in qrb:
            r.wait()
        ownB = pl.ds(pB * CHUNK, CHUNK)
        acc_b[ownB, :] = (acc_b[ownB, :] + qb0[...] + qb1[...] + qb2[...])

        gqa = quad_ag_rdmas(streams[0])
        gqb = quad_ag_rdmas(streams[1])
        for r in gqa + gqb:
            r.start()
        for r in gqa + gqb:
            r.wait()

        for j in (2, 3):
            ga = lin_ag_rdma(streams[0], j)
            gb = lin_ag_rdma(streams[1], j)
            ga.start()
            gb.start()
            ga.wait()
            gb.wait()

        ga = lin_ag_rdma(streams[0], 4)
        gb = lin_ag_rdma(streams[1], 4)
        ga.start()
        gb.start()

        scale = (Dh + Dr) ** -0.5
        nt = (((1,), (1,)), ((), ()))

        def scores_pass(r0):
            rows = pl.ds(r0, S)
            kr_b = kr_ref[rows, :]
            for hh in range(H):
                q_bh = q_ref[rows, hh * Dh:(hh + 1) * Dh]
                qr_bh = qr_ref[rows, hh * Dr:(hh + 1) * Dr]
                k_bh = acc_a[rows, hh * Dh:(hh + 1) * Dh]
                scores = (
                    lax.dot_general(q_bh, k_bh, nt,
                                    preferred_element_type=jnp.float32)
                    + lax.dot_general(qr_bh, kr_b, nt,
                                      preferred_element_type=jnp.float32)
                ) * scale
                m = jnp.max(scores, axis=-1, keepdims=True)
                pr = jnp.exp(scores - m)
                pr = pr / jnp.sum(pr, axis=-1, keepdims=True)
                probs_ref[rows, hh * S:(hh + 1) * S] = pr.astype(jnp.bfloat16)

        scores_pass(x_ * S)

        ga.wait()
        gb.wait()

        scores_pass((1 - x_) * S)

        for b in range(B):
            r0 = b * S
            o_b = jnp.zeros((S, D), jnp.float32)
            for hh in range(H):
                v_bh = acc_b[r0:r0 + S, hh * Dh:(hh + 1) * Dh]
                pr = probs_ref[r0:r0 + S, hh * S:(hh + 1) * S]
                o_bh = jnp.dot(pr, v_bh, preferred_element_type=jnp.float32)
                o_b = o_b + jnp.dot(
                    o_bh.astype(jnp.bfloat16),
                    wo_ref[hh * Dh:(hh + 1) * Dh, :].astype(jnp.bfloat16),
                    preferred_element_type=jnp.float32)
            out_ref[r0:r0 + S, :] = o_b

    lin_stage_shapes = [pltpu.VMEM((r, D), jnp.bfloat16)
                        for r in (256, 128, 64)] * 2
    quad_stage_shapes = [pltpu.VMEM((CHUNK, D), jnp.bfloat16)] * 6

    out = pl.pallas_call(
        body,
        out_shape=jax.ShapeDtypeStruct((ROWS, D), jnp.float32),
        in_specs=[pl.BlockSpec(memory_space=pltpu.VMEM)] * 8,
        out_specs=pl.BlockSpec(memory_space=pltpu.VMEM),
        scratch_shapes=[
            pltpu.VMEM((ROWS, D), jnp.bfloat16),
            pltpu.VMEM((ROWS, D), jnp.bfloat16),
            *lin_stage_shapes,
            *quad_stage_shapes,
            pltpu.VMEM((ROWS, D), jnp.bfloat16),
            pltpu.VMEM((ROWS, H * Dr), jnp.bfloat16),
            pltpu.VMEM((ROWS, Dr), jnp.bfloat16),
            pltpu.VMEM((ROWS, H * S), jnp.bfloat16),
            pltpu.SemaphoreType.DMA((6,)),
            pltpu.SemaphoreType.DMA((6,)),
            pltpu.SemaphoreType.DMA((6,)),
            pltpu.SemaphoreType.DMA((6,)),
            pltpu.SemaphoreType.DMA((6,)),
            pltpu.SemaphoreType.DMA((6,)),
            pltpu.SemaphoreType.DMA((6,)),
            pltpu.SemaphoreType.DMA((6,)),
        ],
        compiler_params=pltpu.CompilerParams(collective_id=0),
    )(x2, Wdkv, Wuk, Wuv, Wq, Wqr, Wkr, Wo)

    return out.reshape(B, S, D)


# device time: 69749 ns/iter; 2.8906x vs baseline; 1.0604x over previous
import jax
import jax.numpy as jnp
from jax import lax
from jax.experimental import pallas as pl
from jax.experimental.pallas import tpu as pltpu

N_DEV = 32
B, S, H, Dh, Dr = 2, 256, 16, 64, 32
D = 1024
ROWS = B * S
CHUNK = ROWS // N_DEV


def kernel(x, Wdkv, Wuk, Wuv, Wq, Wqr, Wkr, Wo):
    x2 = x.reshape(ROWS, D)

    def body(x_ref, wdkv_ref, wuk_ref, wuv_ref, wq_ref, wqr_ref, wkr_ref,
             wo_ref, out_ref, acc_a, acc_b,
             la0, la1, la2, lb0, lb1, lb2,
             qa0, qa1, qa2, qb0, qb1, qb2,
             q_ref, qr_ref, kr_ref, probs_ref,
             rs_send, rs_recv, ag_send, ag_recv,
             qrs_send, qrs_recv, qag_send, qag_recv):
        my = lax.axis_index("i")

        z = my // 8
        o = my % 8
        y = o // 2
        x_ = (y + o) % 2

        def lof(px, py, pz):
            return pz * 8 + py * 2 + (px + py) % 2

        y_1 = y + 1 - 2 * (y % 2)
        z_1 = z + 1 - 2 * (z % 2)
        y_2 = (y + 2) % 4
        z_2 = (z + 2) % 4
        z_3 = (z_1 + 2) % 4

        pA = x_ * 16 + (y % 2) * 8 + (z % 2) * 4 + (y // 2) * 2 + z // 2
        A_lin = [lof(1 - x_, y, z), lof(x_, y_1, z), lof(x_, y, z_1)]
        subA = (y // 2) * 2 + z // 2
        A_quad = [
            (lof(x_, y_2, z), (y_2 // 2) * 2 + z // 2),
            (lof(x_, y, z_2), (y // 2) * 2 + z_2 // 2),
            (lof(x_, y_2, z_2), (y_2 // 2) * 2 + z_2 // 2),
        ]

        pB = (y % 2) * 16 + x_ * 8 + (y // 2) * 4 + (z % 2) * 2 + z // 2
        B_lin = [lof(x_, y_1, z), lof(1 - x_, y, z), lof(x_, y_2, z)]
        subB = (z % 2) * 2 + z // 2
        B_quad = [
            (lof(x_, y, z_1), (z_1 % 2) * 2 + z_1 // 2),
            (lof(x_, y, z_2), (z_2 % 2) * 2 + z_2 // 2),
            (lof(x_, y, z_3), (z_3 % 2) * 2 + z_3 // 2),
        ]

        streams = [
            dict(acc=acc_a, p=pA, lin=A_lin, lst=[la0, la1, la2],
                 qst=[qa0, qa1, qa2], quad=A_quad, sub=subA, so=0),
            dict(acc=acc_b, p=pB, lin=B_lin, lst=[lb0, lb1, lb2],
                 qst=[qb0, qb1, qb2], quad=B_quad, sub=subB, so=3),
        ]

        def lin_rs_rdma(st, k):
            h = 16 >> k
            rows = h * CHUNK
            qq = st["p"] // h
            sent = (qq + 1 - 2 * (qq % 2)) * rows
            return pltpu.make_async_remote_copy(
                src_ref=st["acc"].at[pl.ds(sent, rows), :],
                dst_ref=st["lst"][k].at[...],
                send_sem=rs_send.at[st["so"] + k],
                recv_sem=rs_recv.at[st["so"] + k],
                device_id=(st["lin"][k],),
                device_id_type=pl.DeviceIdType.MESH,
            )

        def lin_rs_acc(st, k):
            h = 16 >> k
            rows = h * CHUNK
            kept = (st["p"] // h) * rows
            st["acc"][pl.ds(kept, rows), :] = (
                st["acc"][pl.ds(kept, rows), :] + st["lst"][k][...])

        def lin_ag_rdma(st, j):
            blk = 1 << j
            rows = blk * CHUNK
            own = (st["p"] // blk) * rows
            return pltpu.make_async_remote_copy(
                src_ref=st["acc"].at[pl.ds(own, rows), :],
                dst_ref=st["acc"].at[pl.ds(own, rows), :],
                send_sem=ag_send.at[st["so"] + (4 - j)],
                recv_sem=ag_recv.at[st["so"] + (4 - j)],
                device_id=(st["lin"][4 - j],),
                device_id_type=pl.DeviceIdType.MESH,
            )

        def quad_rs_rdmas(st):
            q4 = st["p"] // 4
            return [pltpu.make_async_remote_copy(
                src_ref=st["acc"].at[pl.ds((q4 * 4 + ps) * CHUNK, CHUNK), :],
                dst_ref=st["qst"][i].at[...],
                send_sem=qrs_send.at[st["so"] + i],
                recv_sem=qrs_recv.at[st["so"] + i],
                device_id=(prt,),
                device_id_type=pl.DeviceIdType.MESH,
            ) for i, (prt, ps) in enumerate(st["quad"])]

        def quad_ag_rdmas(st):
            own = pl.ds(st["p"] * CHUNK, CHUNK)
            return [pltpu.make_async_remote_copy(
                src_ref=st["acc"].at[own, :],
                dst_ref=st["acc"].at[own, :],
                send_sem=qag_send.at[st["so"] + i],
                recv_sem=qag_recv.at[st["so"] + i],
                device_id=(prt,),
                device_id_type=pl.DeviceIdType.MESH,
            ) for i, (prt, _) in enumerate(st["quad"])]

        xb = x_ref[...].astype(jnp.bfloat16)

        c = jnp.dot(xb, wdkv_ref[...].astype(jnp.bfloat16),
                    preferred_element_type=jnp.float32).astype(jnp.bfloat16)
        kp = jnp.dot(c, wuk_ref[...].astype(jnp.bfloat16),
                     preferred_element_type=jnp.float32)
        vp = jnp.dot(c, wuv_ref[...].astype(jnp.bfloat16),
                     preferred_element_type=jnp.float32)
        acc_a[...] = kp.astype(jnp.bfloat16)
        acc_b[...] = vp.astype(jnp.bfloat16)

        bar_partners = [lof(1 - x_, y, z), lof(x_, y_1, z), lof(x_, y, z_1),
                        lof(x_, y_2, z), lof(x_, y, z_2),
                        lof(x_, y_2, z_2), lof(x_, y, z_3)]
        barrier = pltpu.get_barrier_semaphore()
        for prt in bar_partners:
            pl.semaphore_signal(barrier, inc=1, device_id=(prt,),
                                device_id_type=pl.DeviceIdType.MESH)
        pl.semaphore_wait(barrier, len(bar_partners))

        ra = lin_rs_rdma(streams[0], 0)
        rb = lin_rs_rdma(streams[1], 0)
        ra.start()
        rb.start()
        q_ref[...] = jnp.dot(xb, wq_ref[...].astype(jnp.bfloat16),
                             preferred_element_type=jnp.float32
                             ).astype(jnp.bfloat16)
        qr_ref[...] = jnp.dot(xb, wqr_ref[...].astype(jnp.bfloat16),
                              preferred_element_type=jnp.float32
                              ).astype(jnp.bfloat16)
        kr_ref[...] = jnp.dot(xb, wkr_ref[...].astype(jnp.bfloat16),
                              preferred_element_type=jnp.float32
                              ).astype(jnp.bfloat16)
        ra.wait()
        lin_rs_acc(streams[0], 0)
        rb.wait()
        lin_rs_acc(streams[1], 0)

        for k in (1, 2):
            ra = lin_rs_rdma(streams[0], k)
            rb = lin_rs_rdma(streams[1], k)
            ra.start()
            rb.start()
            ra.wait()
            lin_rs_acc(streams[0], k)
            rb.wait()
            lin_rs_acc(streams[1], k)

        qra = quad_rs_rdmas(streams[0])
        qrb = quad_rs_rdmas(streams[1])
        for r in qra + qrb:
            r.start()
        for r in qra:
            r.wait()
        ownA = pl.ds(pA * CHUNK, CHUNK)
        acc_a[ownA, :] = (acc_a[ownA, :] + qa0[...] + qa1[...] + qa2[...])
        for r in qrb:
            r.wait()
        ownB = pl.ds(pB * CHUNK, CHUNK)
        acc_b[ownB, :] = (acc_b[ownB, :] + qb0[...] + qb1[...] + qb2[...])

        gqa = quad_ag_rdmas(streams[0])
        gqb = quad_ag_rdmas(streams[1])
        for r in gqa + gqb:
            r.start()
        for r in gqa + gqb:
            r.wait()

        for j in (2, 3):
            ga = lin_ag_rdma(streams[0], j)
            gb = lin_ag_rdma(streams[1], j)
            ga.start()
            gb.start()
            ga.wait()
            gb.wait()

        ga = lin_ag_rdma(streams[0], 4)
        gb = lin_ag_rdma(streams[1], 4)
        ga.start()
        gb.start()

        scale = (Dh + Dr) ** -0.5
        nt = (((1,), (1,)), ((), ()))

        def scores_pass(r0):
            rows = pl.ds(r0, S)
            kr_b = kr_ref[rows, :]
            for hh in range(H):
                q_bh = q_ref[rows, hh * Dh:(hh + 1) * Dh]
                qr_bh = qr_ref[rows, hh * Dr:(hh + 1) * Dr]
                k_bh = acc_a[rows, hh * Dh:(hh + 1) * Dh]
                scores = (
                    lax.dot_general(q_bh, k_bh, nt,
                                    preferred_element_type=jnp.float32)
                    + lax.dot_general(qr_bh, kr_b, nt,
                                      preferred_element_type=jnp.float32)
                ) * scale
                m = jnp.max(scores, axis=-1, keepdims=True)
                pr = jnp.exp(scores - m)
                pr = pr / jnp.sum(pr, axis=-1, keepdims=True)
                probs_ref[rows, hh * S:(hh + 1) * S] = pr.astype(jnp.bfloat16)

        scores_pass(x_ * S)

        ga.wait()
        scores_pass((1 - x_) * S)
        gb.wait()

        wo_b = wo_ref[...].astype(jnp.bfloat16)
        for b in range(B):
            r0 = b * S
            o_parts = []
            for hh in range(H):
                v_bh = acc_b[r0:r0 + S, hh * Dh:(hh + 1) * Dh]
                pr = probs_ref[r0:r0 + S, hh * S:(hh + 1) * S]
                o_parts.append(
                    jnp.dot(pr, v_bh, preferred_element_type=jnp.float32
                            ).astype(jnp.bfloat16))
            o_cat = jnp.concatenate(o_parts, axis=1)
            out_ref[r0:r0 + S, :] = jnp.dot(
                o_cat, wo_b, preferred_element_type=jnp.float32)

    lin_stage_shapes = [pltpu.VMEM((r, D), jnp.bfloat16)
                        for r in (256, 128, 64)] * 2
    quad_stage_shapes = [pltpu.VMEM((CHUNK, D), jnp.bfloat16)] * 6

    out = pl.pallas_call(
        body,
        out_shape=jax.ShapeDtypeStruct((ROWS, D), jnp.float32),
        in_specs=[pl.BlockSpec(memory_space=pltpu.VMEM)] * 8,
        out_specs=pl.BlockSpec(memory_space=pltpu.VMEM),
        scratch_shapes=[
            pltpu.VMEM((ROWS, D), jnp.bfloat16),
            pltpu.VMEM((ROWS, D), jnp.bfloat16),
            *lin_stage_shapes,
            *quad_stage_shapes,
            pltpu.VMEM((ROWS, D), jnp.bfloat16),
            pltpu.VMEM((ROWS, H * Dr), jnp.bfloat16),
            pltpu.VMEM((ROWS, Dr), jnp.bfloat16),
            pltpu.VMEM((ROWS, H * S), jnp.bfloat16),
            pltpu.SemaphoreType.DMA((6,)),
            pltpu.SemaphoreType.DMA((6,)),
            pltpu.SemaphoreType.DMA((6,)),
            pltpu.SemaphoreType.DMA((6,)),
            pltpu.SemaphoreType.DMA((6,)),
            pltpu.SemaphoreType.DMA((6,)),
            pltpu.SemaphoreType.DMA((6,)),
            pltpu.SemaphoreType.DMA((6,)),
        ],
        compiler_params=pltpu.CompilerParams(collective_id=0),
    )(x2, Wdkv, Wuk, Wuv, Wq, Wqr, Wkr, Wo)

    return out.reshape(B, S, D)


# device time: 64567 ns/iter; 3.1226x vs baseline; 1.0803x over previous
import jax
import jax.numpy as jnp
from jax import lax
from jax.experimental import pallas as pl
from jax.experimental.pallas import tpu as pltpu

N_DEV = 32
B, S, H, Dh, Dr = 2, 256, 16, 64, 32
D = 1024
ROWS = B * S
CHUNK = ROWS // N_DEV


def kernel(x, Wdkv, Wuk, Wuv, Wq, Wqr, Wkr, Wo):
    x2 = x.reshape(ROWS, D)

    def body(x_ref, wdkv_ref, wuk_ref, wuv_ref, wq_ref, wqr_ref, wkr_ref,
             wo_ref, out_ref, acc_a, acc_b,
             la0, la1, la2, lb0, lb1, lb2,
             qa0, qa1, qa2, qb0, qb1, qb2,
             q_ref, qr_ref, kr_ref, probs_ref,
             rs_send, rs_recv, ag_send, ag_recv,
             qrs_send, qrs_recv, qag_send, qag_recv,
             h0_send, h0_recv):
        my = lax.axis_index("i")

        z = my // 8
        o = my % 8
        y = o // 2
        x_ = (y + o) % 2

        def lof(px, py, pz):
            return pz * 8 + py * 2 + (px + py) % 2

        y_1 = y + 1 - 2 * (y % 2)
        z_1 = z + 1 - 2 * (z % 2)
        y_2 = (y + 2) % 4
        z_2 = (z + 2) % 4
        z_3 = (z_1 + 2) % 4

        pA = x_ * 16 + (y % 2) * 8 + (y // 2) * 4 + (z % 2) * 2 + z // 2
        A_lin = [lof(1 - x_, y, z), lof(x_, y_1, z), lof(x_, y_2, z)]
        A_quad = [
            (lof(x_, y, z_1), (z_1 % 2) * 2 + z_1 // 2),
            (lof(x_, y, z_2), (z_2 % 2) * 2 + z_2 // 2),
            (lof(x_, y, z_3), (z_3 % 2) * 2 + z_3 // 2),
        ]

        pB = (y % 2) * 16 + (z % 2) * 8 + (z // 2) * 4 + x_ * 2 + y // 2
        B_lin = [lof(x_, y_1, z), lof(x_, y, z_1), lof(x_, y, z_2)]
        B_quad = [
            (lof(1 - x_, y, z), (1 - x_) * 2 + y // 2),
            (lof(x_, y_2, z), x_ * 2 + y_2 // 2),
            (lof(1 - x_, y_2, z), (1 - x_) * 2 + y_2 // 2),
        ]

        streams = [
            dict(acc=acc_a, p=pA, lin=A_lin, lst=[la0, la1, la2],
                 qst=[qa0, qa1, qa2], quad=A_quad, so=0, ho=0),
            dict(acc=acc_b, p=pB, lin=B_lin, lst=[lb0, lb1, lb2],
                 qst=[qb0, qb1, qb2], quad=B_quad, so=3, ho=2),
        ]

        def lin_rs_rdma(st, k):
            h = 16 >> k
            rows = h * CHUNK
            qq = st["p"] // h
            sent = (qq + 1 - 2 * (qq % 2)) * rows
            return pltpu.make_async_remote_copy(
                src_ref=st["acc"].at[pl.ds(sent, rows), :],
                dst_ref=st["lst"][k].at[...],
                send_sem=rs_send.at[st["so"] + k],
                recv_sem=rs_recv.at[st["so"] + k],
                device_id=(st["lin"][k],),
                device_id_type=pl.DeviceIdType.MESH,
            )

        def lin_rs_acc(st, k):
            h = 16 >> k
            rows = h * CHUNK
            kept = (st["p"] // h) * rows
            st["acc"][pl.ds(kept, rows), :] = (
                st["acc"][pl.ds(kept, rows), :] + st["lst"][k][...])

        def half0_rdmas(st):
            p = st["p"]
            kept = (p // 16) * 256
            sent = 256 - kept
            pP = p + 16 - 32 * (p // 16)
            q1p = pP // 8
            p_s1 = (q1p + 1 - 2 * (q1p % 2)) * 128
            other = 2 * sent + 128 - p_s1
            mk = lambda src0, dst0, i: pltpu.make_async_remote_copy(
                src_ref=st["acc"].at[pl.ds(src0, 128), :],
                dst_ref=st["lst"][0].at[pl.ds(dst0, 128), :],
                send_sem=h0_send.at[st["ho"] + i],
                recv_sem=h0_recv.at[st["ho"] + i],
                device_id=(st["lin"][0],),
                device_id_type=pl.DeviceIdType.MESH,
            )
            return mk(p_s1, p_s1 - sent, 0), mk(other, other - sent, 1)

        def half0_acc(st, first):
            p = st["p"]
            kept = (p // 16) * 256
            q1 = p // 8
            my_s1 = (q1 + 1 - 2 * (q1 % 2)) * 128
            my_o = 2 * kept + 128 - my_s1
            r0 = my_s1 if first else my_o
            st["acc"][pl.ds(r0, 128), :] = (
                st["acc"][pl.ds(r0, 128), :]
                + st["lst"][0][pl.ds(r0 - kept, 128), :])

        def lin_ag_rdma(st, j):
            blk = 1 << j
            rows = blk * CHUNK
            own = (st["p"] // blk) * rows
            return pltpu.make_async_remote_copy(
                src_ref=st["acc"].at[pl.ds(own, rows), :],
                dst_ref=st["acc"].at[pl.ds(own, rows), :],
                send_sem=ag_send.at[st["so"] + (4 - j)],
                recv_sem=ag_recv.at[st["so"] + (4 - j)],
                device_id=(st["lin"][4 - j],),
                device_id_type=pl.DeviceIdType.MESH,
            )

        def quad_rs_rdmas(st):
            q4 = st["p"] // 4
            return [pltpu.make_async_remote_copy(
                src_ref=st["acc"].at[pl.ds((q4 * 4 + ps) * CHUNK, CHUNK), :],
                dst_ref=st["qst"][i].at[...],
                send_sem=qrs_send.at[st["so"] + i],
                recv_sem=qrs_recv.at[st["so"] + i],
                device_id=(prt,),
                device_id_type=pl.DeviceIdType.MESH,
            ) for i, (prt, ps) in enumerate(st["quad"])]

        def quad_ag_rdmas(st):
            own = pl.ds(st["p"] * CHUNK, CHUNK)
            return [pltpu.make_async_remote_copy(
                src_ref=st["acc"].at[own, :],
                dst_ref=st["acc"].at[own, :],
                send_sem=qag_send.at[st["so"] + i],
                recv_sem=qag_recv.at[st["so"] + i],
                device_id=(prt,),
                device_id_type=pl.DeviceIdType.MESH,
            ) for i, (prt, _) in enumerate(st["quad"])]

        bar_partners = [lof(1 - x_, y, z), lof(x_, y_1, z), lof(x_, y_2, z),
                        lof(x_, y, z_1), lof(x_, y, z_2), lof(x_, y, z_3),
                        lof(1 - x_, y_2, z)]
        barrier = pltpu.get_barrier_semaphore()
        for prt in bar_partners:
            pl.semaphore_signal(barrier, inc=1, device_id=(prt,),
                                device_id_type=pl.DeviceIdType.MESH)

        xb = x_ref[...].astype(jnp.bfloat16)

        c = jnp.dot(xb, wdkv_ref[...].astype(jnp.bfloat16),
                    preferred_element_type=jnp.float32).astype(jnp.bfloat16)
        kp = jnp.dot(c, wuk_ref[...].astype(jnp.bfloat16),
                     preferred_element_type=jnp.float32)
        vp = jnp.dot(c, wuv_ref[...].astype(jnp.bfloat16),
                     preferred_element_type=jnp.float32)
        acc_a[...] = kp.astype(jnp.bfloat16)
        acc_b[...] = vp.astype(jnp.bfloat16)

        pl.semaphore_wait(barrier, len(bar_partners))

        a0a, a0b = half0_rdmas(streams[0])
        b0a, b0b = half0_rdmas(streams[1])
        a0a.start()
        a0b.start()
        b0a.start()
        b0b.start()
        q_ref[...] = jnp.dot(xb, wq_ref[...].astype(jnp.bfloat16),
                             preferred_element_type=jnp.float32
                             ).astype(jnp.bfloat16)
        qr_ref[...] = jnp.dot(xb, wqr_ref[...].astype(jnp.bfloat16),
                              preferred_element_type=jnp.float32
                              ).astype(jnp.bfloat16)
        kr_ref[...] = jnp.dot(xb, wkr_ref[...].astype(jnp.bfloat16),
                              preferred_element_type=jnp.float32
                              ).astype(jnp.bfloat16)
        a0a.wait()
        half0_acc(streams[0], True)
        ra = lin_rs_rdma(streams[0], 1)
        ra.start()
        b0a.wait()
        half0_acc(streams[1], True)
        rb = lin_rs_rdma(streams[1], 1)
        rb.start()
        a0b.wait()
        half0_acc(streams[0], False)
        b0b.wait()
        half0_acc(streams[1], False)

        ra.wait()
        lin_rs_acc(streams[0], 1)
        ra2 = lin_rs_rdma(streams[0], 2)
        ra2.start()
        rb.wait()
        lin_rs_acc(streams[1], 1)
        rb2 = lin_rs_rdma(streams[1], 2)
        rb2.start()
        ra2.wait()
        lin_rs_acc(streams[0], 2)
        rb2.wait()
        lin_rs_acc(streams[1], 2)

        qra = quad_rs_rdmas(streams[0])
        qrb = quad_rs_rdmas(streams[1])
        for r in qra + qrb:
            r.start()
        for r in qra:
            r.wait()
        ownA = pl.ds(pA * CHUNK, CHUNK)
        acc_a[ownA, :] = (acc_a[ownA, :] + qa0[...] + qa1[...] + qa2[...])
        for r in qrb:
            r.wait()
        ownB = pl.ds(pB * CHUNK, CHUNK)
        acc_b[ownB, :] = (acc_b[ownB, :] + qb0[...] + qb1[...] + qb2[...])

        gqa = quad_ag_rdmas(streams[0])
        gqb = quad_ag_rdmas(streams[1])
        for r in gqa + gqb:
            r.start()
        for r in gqa + gqb:
            r.wait()

        for j in (2, 3):
            ga = lin_ag_rdma(streams[0], j)
            gb = lin_ag_rdma(streams[1], j)
            ga.start()
            gb.start()
            ga.wait()
            gb.wait()

        ga = lin_ag_rdma(streams[0], 4)
        gb = lin_ag_rdma(streams[1], 4)
        ga.start()
        gb.start()

        scale = (Dh + Dr) ** -0.5
        nt = (((1,), (1,)), ((), ()))

        def scores_pass(r0):
            rows = pl.ds(r0, S)
            kr_b = kr_ref[rows, :]
            for hh in range(H):
                q_bh = q_ref[rows, hh * Dh:(hh + 1) * Dh]
                qr_bh = qr_ref[rows, hh * Dr:(hh + 1) * Dr]
                k_bh = acc_a[rows, hh * Dh:(hh + 1) * Dh]
                scores = (
                    lax.dot_general(q_bh, k_bh, nt,
                                    preferred_element_type=jnp.float32)
                    + lax.dot_general(qr_bh, kr_b, nt,
                                      preferred_element_type=jnp.float32)
                ) * scale
                m = jnp.max(scores, axis=-1, keepdims=True)
                pr = jnp.exp(scores - m)
                pr = pr / jnp.sum(pr, axis=-1, keepdims=True)
                probs_ref[rows, hh * S:(hh + 1) * S] = pr.astype(jnp.bfloat16)

        scores_pass(x_ * S)

        ga.wait()
        scores_pass((1 - x_) * S)
        gb.wait()

        wo_b = wo_ref[...].astype(jnp.bfloat16)
        for b in range(B):
            r0 = b * S
            o_parts = []
            for hh in range(H):
                v_bh = acc_b[r0:r0 + S, hh * Dh:(hh + 1) * Dh]
                pr = probs_ref[r0:r0 + S, hh * S:(hh + 1) * S]
                o_parts.append(
                    jnp.dot(pr, v_bh, preferred_element_type=jnp.float32
                            ).astype(jnp.bfloat16))
            o_cat = jnp.concatenate(o_parts, axis=1)
            out_ref[r0:r0 + S, :] = jnp.dot(
                o_cat, wo_b, preferred_element_type=jnp.float32)

    lin_stage_shapes = [pltpu.VMEM((r, D), jnp.bfloat16)
                        for r in (256, 128, 64)] * 2
    quad_stage_shapes = [pltpu.VMEM((CHUNK, D), jnp.bfloat16)] * 6

    out = pl.pallas_call(
        body,
        out_shape=jax.ShapeDtypeStruct((ROWS, D), jnp.float32),
        in_specs=[pl.BlockSpec(memory_space=pltpu.VMEM)] * 8,
        out_specs=pl.BlockSpec(memory_space=pltpu.VMEM),
        scratch_shapes=[
            pltpu.VMEM((ROWS, D), jnp.bfloat16),
            pltpu.VMEM((ROWS, D), jnp.bfloat16),
            *lin_stage_shapes,
            *quad_stage_shapes,
            pltpu.VMEM((ROWS, D), jnp.bfloat16),
            pltpu.VMEM((ROWS, H * Dr), jnp.bfloat16),
            pltpu.VMEM((ROWS, Dr), jnp.bfloat16),
            pltpu.VMEM((ROWS, H * S), jnp.bfloat16),
            pltpu.SemaphoreType.DMA((6,)),
            pltpu.SemaphoreType.DMA((6,)),
            pltpu.SemaphoreType.DMA((6,)),
            pltpu.SemaphoreType.DMA((6,)),
            pltpu.SemaphoreType.DMA((6,)),
            pltpu.SemaphoreType.DMA((6,)),
            pltpu.SemaphoreType.DMA((6,)),
            pltpu.SemaphoreType.DMA((6,)),
            pltpu.SemaphoreType.DMA((4,)),
            pltpu.SemaphoreType.DMA((4,)),
        ],
        compiler_params=pltpu.CompilerParams(collective_id=0),
    )(x2, Wdkv, Wuk, Wuv, Wq, Wqr, Wkr, Wo)

    return out.reshape(B, S, D)


# device time: 63112 ns/iter; 3.1946x vs baseline; 1.0231x over previous
import jax
import jax.numpy as jnp
from jax import lax
from jax.experimental import pallas as pl
from jax.experimental.pallas import tpu as pltpu

N_DEV = 32
B, S, H, Dh, Dr = 2, 256, 16, 64, 32
D = 1024
ROWS = B * S
CHUNK = ROWS // N_DEV


def kernel(x, Wdkv, Wuk, Wuv, Wq, Wqr, Wkr, Wo):
    x2 = x.reshape(ROWS, D)

    def body(x_ref, wdkv_ref, wuk_ref, wuv_ref, wq_ref, wqr_ref, wkr_ref,
             wo_ref, out_ref, acc_a, acc_b,
             la0, la1, la2, lb0, lb1, lb2,
             qa0, qa1, qa2, qb0, qb1, qb2,
             q_ref, qr_ref, kr_ref, probs_ref,
             rs_send, rs_recv, ag_send, ag_recv,
             qrs_send, qrs_recv, qag_send, qag_recv,
             h0_send, h0_recv):
        my = lax.axis_index("i")

        z = my // 8
        o = my % 8
        y = o // 2
        x_ = (y + o) % 2

        def lof(px, py, pz):
            return pz * 8 + py * 2 + (px + py) % 2

        y_1 = y + 1 - 2 * (y % 2)
        z_1 = z + 1 - 2 * (z % 2)
        y_2 = (y + 2) % 4
        z_2 = (z + 2) % 4
        z_3 = (z_1 + 2) % 4

        pA = x_ * 16 + (y % 2) * 8 + (y // 2) * 4 + (z % 2) * 2 + z // 2
        A_lin = [lof(1 - x_, y, z), lof(x_, y_1, z), lof(x_, y_2, z)]
        A_quad = [
            (lof(x_, y, z_1), (z_1 % 2) * 2 + z_1 // 2),
            (lof(x_, y, z_2), (z_2 % 2) * 2 + z_2 // 2),
            (lof(x_, y, z_3), (z_3 % 2) * 2 + z_3 // 2),
        ]

        pB = (y % 2) * 16 + (z % 2) * 8 + (z // 2) * 4 + x_ * 2 + y // 2
        B_lin = [lof(x_, y_1, z), lof(x_, y, z_1), lof(x_, y, z_2)]
        B_quad = [
            (lof(1 - x_, y, z), (1 - x_) * 2 + y // 2),
            (lof(x_, y_2, z), x_ * 2 + y_2 // 2),
            (lof(1 - x_, y_2, z), (1 - x_) * 2 + y_2 // 2),
        ]

        streams = [
            dict(acc=acc_a, p=pA, lin=A_lin, lst=[la0, la1, la2],
                 qst=[qa0, qa1, qa2], quad=A_quad, so=0, ho=0),
            dict(acc=acc_b, p=pB, lin=B_lin, lst=[lb0, lb1, lb2],
                 qst=[qb0, qb1, qb2], quad=B_quad, so=3, ho=2),
        ]

        def lin_rs_rdma(st, k):
            h = 16 >> k
            rows = h * CHUNK
            qq = st["p"] // h
            sent = (qq + 1 - 2 * (qq % 2)) * rows
            return pltpu.make_async_remote_copy(
                src_ref=st["acc"].at[pl.ds(sent, rows), :],
                dst_ref=st["lst"][k].at[...],
                send_sem=rs_send.at[st["so"] + k],
                recv_sem=rs_recv.at[st["so"] + k],
                device_id=(st["lin"][k],),
                device_id_type=pl.DeviceIdType.MESH,
            )

        def lin_rs_acc(st, k):
            h = 16 >> k
            rows = h * CHUNK
            kept = (st["p"] // h) * rows
            st["acc"][pl.ds(kept, rows), :] = (
                st["acc"][pl.ds(kept, rows), :] + st["lst"][k][...])

        def half0_rdmas(st):
            p = st["p"]
            kept = (p // 16) * 256
            sent = 256 - kept
            pP = p + 16 - 32 * (p // 16)
            q1p = pP // 8
            p_s1 = (q1p + 1 - 2 * (q1p % 2)) * 128
            other = 2 * sent + 128 - p_s1
            mk = lambda src0, dst0, i: pltpu.make_async_remote_copy(
                src_ref=st["acc"].at[pl.ds(src0, 128), :],
                dst_ref=st["lst"][0].at[pl.ds(dst0, 128), :],
                send_sem=h0_send.at[st["ho"] + i],
                recv_sem=h0_recv.at[st["ho"] + i],
                device_id=(st["lin"][0],),
                device_id_type=pl.DeviceIdType.MESH,
            )
            return mk(p_s1, p_s1 - sent, 0), mk(other, other - sent, 1)

        def half0_acc(st, first):
            p = st["p"]
            kept = (p // 16) * 256
            q1 = p // 8
            my_s1 = (q1 + 1 - 2 * (q1 % 2)) * 128
            my_o = 2 * kept + 128 - my_s1
            r0 = my_s1 if first else my_o
            st["acc"][pl.ds(r0, 128), :] = (
                st["acc"][pl.ds(r0, 128), :]
                + st["lst"][0][pl.ds(r0 - kept, 128), :])

        def lin_ag_rdma(st, j):
            blk = 1 << j
            rows = blk * CHUNK
            own = (st["p"] // blk) * rows
            return pltpu.make_async_remote_copy(
                src_ref=st["acc"].at[pl.ds(own, rows), :],
                dst_ref=st["acc"].at[pl.ds(own, rows), :],
                send_sem=ag_send.at[st["so"] + (4 - j)],
                recv_sem=ag_recv.at[st["so"] + (4 - j)],
                device_id=(st["lin"][4 - j],),
                device_id_type=pl.DeviceIdType.MESH,
            )

        def quad_rs_rdmas(st):
            q4 = st["p"] // 4
            return [pltpu.make_async_remote_copy(
                src_ref=st["acc"].at[pl.ds((q4 * 4 + ps) * CHUNK, CHUNK), :],
                dst_ref=st["qst"][i].at[...],
                send_sem=qrs_send.at[st["so"] + i],
                recv_sem=qrs_recv.at[st["so"] + i],
                device_id=(prt,),
                device_id_type=pl.DeviceIdType.MESH,
            ) for i, (prt, ps) in enumerate(st["quad"])]

        def quad_ag_rdmas(st):
            own = pl.ds(st["p"] * CHUNK, CHUNK)
            return [pltpu.make_async_remote_copy(
                src_ref=st["acc"].at[own, :],
                dst_ref=st["acc"].at[own, :],
                send_sem=qag_send.at[st["so"] + i],
                recv_sem=qag_recv.at[st["so"] + i],
                device_id=(prt,),
                device_id_type=pl.DeviceIdType.MESH,
            ) for i, (prt, _) in enumerate(st["quad"])]

        bar_partners = [lof(1 - x_, y, z), lof(x_, y_1, z), lof(x_, y_2, z),
                        lof(x_, y, z_1), lof(x_, y, z_2), lof(x_, y, z_3),
                        lof(1 - x_, y_2, z)]
        barrier = pltpu.get_barrier_semaphore()
        for prt in bar_partners:
            pl.semaphore_signal(barrier, inc=1, device_id=(prt,),
                                device_id_type=pl.DeviceIdType.MESH)

        xb = x_ref[...].astype(jnp.bfloat16)

        c = jnp.dot(xb, wdkv_ref[...].astype(jnp.bfloat16),
                    preferred_element_type=jnp.float32).astype(jnp.bfloat16)
        wuk_b = wuk_ref[...].astype(jnp.bfloat16)
        wuv_b = wuv_ref[...].astype(jnp.bfloat16)

        pl.semaphore_wait(barrier, len(bar_partners))

        acc_a[...] = jnp.dot(c, wuk_b,
                             preferred_element_type=jnp.float32
                             ).astype(jnp.bfloat16)
        acc_b[...] = jnp.dot(c, wuv_b,
                             preferred_element_type=jnp.float32
                             ).astype(jnp.bfloat16)

        a0a, a0b = half0_rdmas(streams[0])
        b0a, b0b = half0_rdmas(streams[1])
        a0a.start()
        a0b.start()
        b0a.start()
        b0b.start()

        scale = (Dh + Dr) ** -0.5
        q_ref[...] = (jnp.dot(xb, wq_ref[...].astype(jnp.bfloat16),
                              preferred_element_type=jnp.float32) * scale
                      ).astype(jnp.bfloat16)
        qr_ref[...] = (jnp.dot(xb, wqr_ref[...].astype(jnp.bfloat16),
                               preferred_element_type=jnp.float32) * scale
                       ).astype(jnp.bfloat16)
        kr_ref[...] = jnp.dot(xb, wkr_ref[...].astype(jnp.bfloat16),
                              preferred_element_type=jnp.float32
                              ).astype(jnp.bfloat16)
        a0a.wait()
        half0_acc(streams[0], True)
        ra = lin_rs_rdma(streams[0], 1)
        ra.start()
        b0a.wait()
        half0_acc(streams[1], True)
        rb = lin_rs_rdma(streams[1], 1)
        rb.start()
        a0b.wait()
        half0_acc(streams[0], False)
        b0b.wait()
        half0_acc(streams[1], False)

        ra.wait()
        lin_rs_acc(streams[0], 1)
        ra2 = lin_rs_rdma(streams[0], 2)
        ra2.start()
        rb.wait()
        lin_rs_acc(streams[1], 1)
        rb2 = lin_rs_rdma(streams[1], 2)
        rb2.start()
        ra2.wait()
        lin_rs_acc(streams[0], 2)
        rb2.wait()
        lin_rs_acc(streams[1], 2)

        qra = quad_rs_rdmas(streams[0])
        qrb = quad_rs_rdmas(streams[1])
        for r in qra + qrb:
            r.start()
        for r in qra:
            r.wait()
        ownA = pl.ds(pA * CHUNK, CHUNK)
        acc_a[ownA, :] = (acc_a[ownA, :] + qa0[...] + qa1[...] + qa2[...])
        for r in qrb:
            r.wait()
        ownB = pl.ds(pB * CHUNK, CHUNK)
        acc_b[ownB, :] = (acc_b[ownB, :] + qb0[...] + qb1[...] + qb2[...])

        gqa = quad_ag_rdmas(streams[0])
        gqb = quad_ag_rdmas(streams[1])
        for r in gqa + gqb:
            r.start()
        for r in gqa + gqb:
            r.wait()

        for j in (2, 3):
            ga = lin_ag_rdma(streams[0], j)
            gb = lin_ag_rdma(streams[1], j)
            ga.start()
            gb.start()
            ga.wait()
            gb.wait()

        ga = lin_ag_rdma(streams[0], 4)
        gb = lin_ag_rdma(streams[1], 4)
        ga.start()
        gb.start()

        nt = (((1,), (1,)), ((), ()))

        def scores_pass(r0):
            rows = pl.ds(r0, S)
            kr_b = kr_ref[rows, :]
            for hh in range(H):
                q_bh = q_ref[rows, hh * Dh:(hh + 1) * Dh]
                qr_bh = qr_ref[rows, hh * Dr:(hh + 1) * Dr]
                k_bh = acc_a[rows, hh * Dh:(hh + 1) * Dh]
                scores = (
                    lax.dot_general(q_bh, k_bh, nt,
                                    preferred_element_type=jnp.float32)
                    + lax.dot_general(qr_bh, kr_b, nt,
                                      preferred_element_type=jnp.float32))
                pr = jnp.exp(scores)
                pr = pr / jnp.sum(pr, axis=-1, keepdims=True)
                probs_ref[rows, hh * S:(hh + 1) * S] = pr.astype(jnp.bfloat16)

        scores_pass(x_ * S)

        ga.wait()
        scores_pass((1 - x_) * S)
        gb.wait()

        wo_b = wo_ref[...].astype(jnp.bfloat16)
        for b in range(B):
            r0 = b * S
            o_parts = []
            for hh in range(H):
                v_bh = acc_b[r0:r0 + S, hh * Dh:(hh + 1) * Dh]
                pr = probs_ref[r0:r0 + S, hh * S:(hh + 1) * S]
                o_parts.append(
                    jnp.dot(pr, v_bh, preferred_element_type=jnp.float32
                            ).astype(jnp.bfloat16))
            o_cat = jnp.concatenate(o_parts, axis=1)
            out_ref[r0:r0 + S, :] = jnp.dot(
                o_cat, wo_b, preferred_element_type=jnp.float32)

    lin_stage_shapes = [pltpu.VMEM((r, D), jnp.bfloat16)
                        for r in (256, 128, 64)] * 2
    quad_stage_shapes = [pltpu.VMEM((CHUNK, D), jnp.bfloat16)] * 6

    out = pl.pallas_call(
        body,
        out_shape=jax.ShapeDtypeStruct((ROWS, D), jnp.float32),
        in_specs=[pl.BlockSpec(memory_space=pltpu.VMEM)] * 8,
        out_specs=pl.BlockSpec(memory_space=pltpu.VMEM),
        scratch_shapes=[
            pltpu.VMEM((ROWS, D), jnp.bfloat16),
            pltpu.VMEM((ROWS, D), jnp.bfloat16),
            *lin_stage_shapes,
            *quad_stage_shapes,
            pltpu.VMEM((ROWS, D), jnp.bfloat16),
            pltpu.VMEM((ROWS, H * Dr), jnp.bfloat16),
            pltpu.VMEM((ROWS, Dr), jnp.bfloat16),
            pltpu.VMEM((ROWS, H * S), jnp.bfloat16),
            pltpu.SemaphoreType.DMA((6,)),
            pltpu.SemaphoreType.DMA((6,)),
            pltpu.SemaphoreType.DMA((6,)),
            pltpu.SemaphoreType.DMA((6,)),
            pltpu.SemaphoreType.DMA((6,)),
            pltpu.SemaphoreType.DMA((6,)),
            pltpu.SemaphoreType.DMA((6,)),
            pltpu.SemaphoreType.DMA((6,)),
            pltpu.SemaphoreType.DMA((4,)),
            pltpu.SemaphoreType.DMA((4,)),
        ],
        compiler_params=pltpu.CompilerParams(collective_id=0),
    )(x2, Wdkv, Wuk, Wuv, Wq, Wqr, Wkr, Wo)

    return out.reshape(B, S, D)
